# baseline (speedup 1.0000x reference)
import functools

import jax
import jax.numpy as jnp
from jax import lax
from jax.experimental import pallas as pl
from jax.experimental.pallas import tpu as pltpu
from jax.experimental.pallas import tpu_sc as plsc

N_NODES = 10000
FEAT = 128
E_EDGES = 320000

NCORES = 2
NSUB = 16
K = 256
CHUNKS_PER_SUB = 40
CHUNKS_PER_CORE = CHUNKS_PER_SUB * NSUB
EP = CHUNKS_PER_SUB * NCORES * NSUB * K

NP = 10240
ROWS_PER_SUB = NP // NSUB
ZROWS = 128
NZ = ROWS_PER_SUB // ZROWS

BLK = 2048
GRID = NP // BLK

_mesh = plsc.VectorSubcoreMesh(core_axis_name="c", subcore_axis_name="s")
_f32 = jnp.float32


@functools.partial(
    pl.kernel,
    out_type=jax.ShapeDtypeStruct((NCORES, NP, FEAT), _f32),
    mesh=_mesh,
    scratch_types=[
        pltpu.VMEM((K,), jnp.int32),
        pltpu.VMEM((K, FEAT), _f32),
        pltpu.VMEM_SHARED((NP, FEAT), _f32),
    ],
)
def _deg_kernel(dst_hbm, ones_hbm, zeros_hbm, out_hbm, didx, ov, acc):
    c = lax.axis_index("c")
    s = lax.axis_index("s")
    pltpu.sync_copy(zeros_hbm, ov.at[pl.ds(0, ZROWS)])

    @pl.loop(0, NZ)
    def _(b):
        pltpu.sync_copy(ov.at[pl.ds(0, ZROWS)],
                        acc.at[pl.ds(s * ROWS_PER_SUB + b * ZROWS, ZROWS)])

    pltpu.sync_copy(ones_hbm, ov)
    plsc.subcore_barrier()
    base = (c * CHUNKS_PER_CORE + s * CHUNKS_PER_SUB) * K

    @pl.loop(0, CHUNKS_PER_SUB)
    def _(j):
        pltpu.sync_copy(dst_hbm.at[pl.ds(base + j * K, K)], didx)
        pltpu.sync_copy(ov, acc.at[didx], add=True)

    plsc.subcore_barrier()

    @pl.loop(0, NZ)
    def _(b):
        r = s * ROWS_PER_SUB + b * ZROWS
        pltpu.sync_copy(acc.at[pl.ds(r, ZROWS)], out_hbm.at[c, pl.ds(r, ZROWS)])


@functools.partial(
    pl.kernel,
    out_type=jax.ShapeDtypeStruct((NCORES, NP, FEAT), _f32),
    mesh=_mesh,
    scratch_types=[
        pltpu.VMEM((K,), jnp.int32),
        pltpu.VMEM((K,), jnp.int32),
        pltpu.VMEM((K, FEAT), _f32),
        pltpu.VMEM_SHARED((NP, FEAT), _f32),
    ],
)
def _edge_kernel(hs_hbm, src_hbm, dst_hbm, zeros_hbm, out_hbm,
                 sidx, didx, rows, acc):
    c = lax.axis_index("c")
    s = lax.axis_index("s")
    pltpu.sync_copy(zeros_hbm, rows.at[pl.ds(0, ZROWS)])

    @pl.loop(0, NZ)
    def _(b):
        pltpu.sync_copy(rows.at[pl.ds(0, ZROWS)],
                        acc.at[pl.ds(s * ROWS_PER_SUB + b * ZROWS, ZROWS)])

    plsc.subcore_barrier()
    base = (c * CHUNKS_PER_CORE + s * CHUNKS_PER_SUB) * K

    @pl.loop(0, CHUNKS_PER_SUB)
    def _(j):
        e = base + j * K
        pltpu.sync_copy(src_hbm.at[pl.ds(e, K)], sidx)
        pltpu.sync_copy(dst_hbm.at[pl.ds(e, K)], didx)
        pltpu.sync_copy(hs_hbm.at[sidx], rows)
        pltpu.sync_copy(rows, acc.at[didx], add=True)

    plsc.subcore_barrier()

    @pl.loop(0, NZ)
    def _(b):
        r = s * ROWS_PER_SUB + b * ZROWS
        pltpu.sync_copy(acc.at[pl.ds(r, ZROWS)], out_hbm.at[c, pl.ds(r, ZROWS)])


def _mm_body(x_ref, w_ref, o_ref):
    o_ref[...] = jnp.dot(x_ref[...], w_ref[...], preferred_element_type=_f32)


def _mm(x, w):
    return pl.pallas_call(
        _mm_body,
        grid=(GRID,),
        in_specs=[pl.BlockSpec((BLK, FEAT), lambda i: (i, 0)),
                  pl.BlockSpec((FEAT, FEAT), lambda i: (0, 0))],
        out_specs=pl.BlockSpec((BLK, FEAT), lambda i: (i, 0)),
        out_shape=jax.ShapeDtypeStruct((NP, FEAT), _f32),
    )(x, w)


def _scale_body(degp_ref, h_ref, hs_ref, dinv_ref):
    dinv = lax.rsqrt(degp_ref[0] + degp_ref[1] + 1.0)
    dinv_ref[...] = dinv
    hs_ref[...] = h_ref[...] * dinv


def _scale(degp, h):
    return pl.pallas_call(
        _scale_body,
        grid=(GRID,),
        in_specs=[pl.BlockSpec((NCORES, BLK, FEAT), lambda i: (0, i, 0)),
                  pl.BlockSpec((BLK, FEAT), lambda i: (i, 0))],
        out_specs=[pl.BlockSpec((BLK, FEAT), lambda i: (i, 0)),
                   pl.BlockSpec((BLK, FEAT), lambda i: (i, 0))],
        out_shape=[jax.ShapeDtypeStruct((NP, FEAT), _f32),
                   jax.ShapeDtypeStruct((NP, FEAT), _f32)],
    )(degp, h)


def _dense2_body(dinv_ref, accp_ref, h1_ref, b1_ref, w2_ref, h2_ref, hs2_ref):
    dinv = dinv_ref[...]
    ap = accp_ref[...]
    out1 = (ap[0] + ap[1]) * dinv + h1_ref[...] * dinv * dinv + b1_ref[...]
    h2 = jnp.dot(out1, w2_ref[...], preferred_element_type=_f32)
    h2_ref[...] = h2
    hs2_ref[...] = h2 * dinv


def _dense2(dinv, accp, h1, b1, w2):
    return pl.pallas_call(
        _dense2_body,
        grid=(GRID,),
        in_specs=[pl.BlockSpec((BLK, FEAT), lambda i: (i, 0)),
                  pl.BlockSpec((NCORES, BLK, FEAT), lambda i: (0, i, 0)),
                  pl.BlockSpec((BLK, FEAT), lambda i: (i, 0)),
                  pl.BlockSpec((1, FEAT), lambda i: (0, 0)),
                  pl.BlockSpec((FEAT, FEAT), lambda i: (0, 0))],
        out_specs=[pl.BlockSpec((BLK, FEAT), lambda i: (i, 0)),
                   pl.BlockSpec((BLK, FEAT), lambda i: (i, 0))],
        out_shape=[jax.ShapeDtypeStruct((NP, FEAT), _f32),
                   jax.ShapeDtypeStruct((NP, FEAT), _f32)],
    )(dinv, accp, h1, b1, w2)


def _final_body(dinv_ref, accp_ref, h2_ref, b2_ref, y_ref):
    dinv = dinv_ref[...]
    ap = accp_ref[...]
    out2 = (ap[0] + ap[1]) * dinv + h2_ref[...] * dinv * dinv + b2_ref[...]
    m = jnp.max(out2, axis=-1, keepdims=True)
    z = out2 - m
    y_ref[...] = z - jnp.log(jnp.sum(jnp.exp(z), axis=-1, keepdims=True))


def _final(dinv, accp, h2, b2):
    return pl.pallas_call(
        _final_body,
        grid=(GRID,),
        in_specs=[pl.BlockSpec((BLK, FEAT), lambda i: (i, 0)),
                  pl.BlockSpec((NCORES, BLK, FEAT), lambda i: (0, i, 0)),
                  pl.BlockSpec((BLK, FEAT), lambda i: (i, 0)),
                  pl.BlockSpec((1, FEAT), lambda i: (0, 0))],
        out_specs=pl.BlockSpec((BLK, FEAT), lambda i: (i, 0)),
        out_shape=jax.ShapeDtypeStruct((NP, FEAT), _f32),
    )(dinv, accp, h2, b2)


def kernel(x, edge_index, W1, b1, W2, b2):
    xp = jnp.zeros((NP, FEAT), _f32).at[:N_NODES].set(x)
    pad = jnp.full((EP - E_EDGES,), N_NODES, jnp.int32)
    src = jnp.concatenate([edge_index[0], pad])
    dst = jnp.concatenate([edge_index[1], pad])
    ones128 = jnp.ones((K, FEAT), _f32)
    zeros128 = jnp.zeros((ZROWS, FEAT), _f32)

    degp = _deg_kernel(dst, ones128, zeros128)
    h1 = _mm(xp, W1)
    hs1, dinv = _scale(degp, h1)
    acc1 = _edge_kernel(hs1, src, dst, zeros128)
    h2, hs2 = _dense2(dinv, acc1, h1, b1.reshape(1, FEAT), W2)
    acc2 = _edge_kernel(hs2, src, dst, zeros128)
    y = _final(dinv, acc2, h2, b2.reshape(1, FEAT))
    return y[:N_NODES]


# speedup vs baseline: 1.5235x; 1.5235x over previous
import functools

import jax
import jax.numpy as jnp
from jax import lax
from jax.experimental import pallas as pl
from jax.experimental.pallas import tpu as pltpu
from jax.experimental.pallas import tpu_sc as plsc

N_NODES = 10000
FEAT = 128
E_EDGES = 320000

NCORES = 2
NSUB = 16
K = 128
CHUNKS_PER_SUB = 79
CHUNKS_PER_CORE = CHUNKS_PER_SUB * NSUB
EP = CHUNKS_PER_SUB * NCORES * NSUB * K

NP = 10240
ROWS_PER_SUB = NP // NSUB
ZROWS = 128
NZ = ROWS_PER_SUB // ZROWS

BLK = 2048
GRID = NP // BLK

_mesh = plsc.VectorSubcoreMesh(core_axis_name="c", subcore_axis_name="s")
_f32 = jnp.float32


@functools.partial(
    pl.kernel,
    out_type=jax.ShapeDtypeStruct((NCORES, NP, FEAT), _f32),
    mesh=_mesh,
    scratch_types=[
        pltpu.VMEM((K,), jnp.int32),
        pltpu.VMEM((K, FEAT), _f32),
        pltpu.VMEM_SHARED((NP, FEAT), _f32),
    ],
)
def _deg_kernel(dst_hbm, ones_hbm, zeros_hbm, out_hbm, didx, ov, acc):
    c = lax.axis_index("c")
    s = lax.axis_index("s")
    pltpu.sync_copy(zeros_hbm, ov.at[pl.ds(0, ZROWS)])

    @pl.loop(0, NZ)
    def _(b):
        pltpu.sync_copy(ov.at[pl.ds(0, ZROWS)],
                        acc.at[pl.ds(s * ROWS_PER_SUB + b * ZROWS, ZROWS)])

    pltpu.sync_copy(ones_hbm, ov)
    plsc.subcore_barrier()
    base = (c * CHUNKS_PER_CORE + s * CHUNKS_PER_SUB) * K

    @pl.loop(0, CHUNKS_PER_SUB)
    def _(j):
        pltpu.sync_copy(dst_hbm.at[pl.ds(base + j * K, K)], didx)
        pltpu.sync_copy(ov, acc.at[didx], add=True)

    plsc.subcore_barrier()

    @pl.loop(0, NZ)
    def _(b):
        r = s * ROWS_PER_SUB + b * ZROWS
        pltpu.sync_copy(acc.at[pl.ds(r, ZROWS)], out_hbm.at[c, pl.ds(r, ZROWS)])


@functools.partial(
    pl.kernel,
    out_type=jax.ShapeDtypeStruct((NCORES, NP, FEAT), _f32),
    mesh=_mesh,
    scratch_types=[
        pltpu.VMEM((K,), jnp.int32),
        pltpu.VMEM((K,), jnp.int32),
        pltpu.VMEM((K, FEAT), _f32),
        pltpu.VMEM_SHARED((NP, FEAT), _f32),
    ],
)
def _edge_kernel(hs_hbm, src_hbm, dst_hbm, zeros_hbm, out_hbm,
                 sidx, didx, rows, acc):
    c = lax.axis_index("c")
    s = lax.axis_index("s")
    pltpu.sync_copy(zeros_hbm, rows.at[pl.ds(0, ZROWS)])

    @pl.loop(0, NZ)
    def _(b):
        pltpu.sync_copy(rows.at[pl.ds(0, ZROWS)],
                        acc.at[pl.ds(s * ROWS_PER_SUB + b * ZROWS, ZROWS)])

    plsc.subcore_barrier()
    base = (c * CHUNKS_PER_CORE + s * CHUNKS_PER_SUB) * K

    @pl.loop(0, CHUNKS_PER_SUB)
    def _(j):
        e = base + j * K
        pltpu.sync_copy(src_hbm.at[pl.ds(e, K)], sidx)
        pltpu.sync_copy(dst_hbm.at[pl.ds(e, K)], didx)
        pltpu.sync_copy(hs_hbm.at[sidx], rows)
        pltpu.sync_copy(rows, acc.at[didx], add=True)

    plsc.subcore_barrier()

    @pl.loop(0, NZ)
    def _(b):
        r = s * ROWS_PER_SUB + b * ZROWS
        pltpu.sync_copy(acc.at[pl.ds(r, ZROWS)], out_hbm.at[c, pl.ds(r, ZROWS)])


def _mm_body(x_ref, w_ref, o_ref):
    o_ref[...] = jnp.dot(x_ref[...], w_ref[...], preferred_element_type=_f32)


def _mm(x, w):
    return pl.pallas_call(
        _mm_body,
        grid=(GRID,),
        in_specs=[pl.BlockSpec((BLK, FEAT), lambda i: (i, 0)),
                  pl.BlockSpec((FEAT, FEAT), lambda i: (0, 0))],
        out_specs=pl.BlockSpec((BLK, FEAT), lambda i: (i, 0)),
        out_shape=jax.ShapeDtypeStruct((NP, FEAT), _f32),
    )(x, w)


def _scale_body(degp_ref, h_ref, hs_ref, dinv_ref):
    dinv = lax.rsqrt(degp_ref[0] + degp_ref[1] + 1.0)
    dinv_ref[...] = dinv
    hs_ref[...] = h_ref[...] * dinv


def _scale(degp, h):
    return pl.pallas_call(
        _scale_body,
        grid=(GRID,),
        in_specs=[pl.BlockSpec((NCORES, BLK, FEAT), lambda i: (0, i, 0)),
                  pl.BlockSpec((BLK, FEAT), lambda i: (i, 0))],
        out_specs=[pl.BlockSpec((BLK, FEAT), lambda i: (i, 0)),
                   pl.BlockSpec((BLK, FEAT), lambda i: (i, 0))],
        out_shape=[jax.ShapeDtypeStruct((NP, FEAT), _f32),
                   jax.ShapeDtypeStruct((NP, FEAT), _f32)],
    )(degp, h)


def _dense2_body(dinv_ref, accp_ref, h1_ref, b1_ref, w2_ref, h2_ref, hs2_ref):
    dinv = dinv_ref[...]
    ap = accp_ref[...]
    out1 = (ap[0] + ap[1]) * dinv + h1_ref[...] * dinv * dinv + b1_ref[...]
    h2 = jnp.dot(out1, w2_ref[...], preferred_element_type=_f32)
    h2_ref[...] = h2
    hs2_ref[...] = h2 * dinv


def _dense2(dinv, accp, h1, b1, w2):
    return pl.pallas_call(
        _dense2_body,
        grid=(GRID,),
        in_specs=[pl.BlockSpec((BLK, FEAT), lambda i: (i, 0)),
                  pl.BlockSpec((NCORES, BLK, FEAT), lambda i: (0, i, 0)),
                  pl.BlockSpec((BLK, FEAT), lambda i: (i, 0)),
                  pl.BlockSpec((1, FEAT), lambda i: (0, 0)),
                  pl.BlockSpec((FEAT, FEAT), lambda i: (0, 0))],
        out_specs=[pl.BlockSpec((BLK, FEAT), lambda i: (i, 0)),
                   pl.BlockSpec((BLK, FEAT), lambda i: (i, 0))],
        out_shape=[jax.ShapeDtypeStruct((NP, FEAT), _f32),
                   jax.ShapeDtypeStruct((NP, FEAT), _f32)],
    )(dinv, accp, h1, b1, w2)


def _final_body(dinv_ref, accp_ref, h2_ref, b2_ref, y_ref):
    dinv = dinv_ref[...]
    ap = accp_ref[...]
    out2 = (ap[0] + ap[1]) * dinv + h2_ref[...] * dinv * dinv + b2_ref[...]
    m = jnp.max(out2, axis=-1, keepdims=True)
    z = out2 - m
    y_ref[...] = z - jnp.log(jnp.sum(jnp.exp(z), axis=-1, keepdims=True))


def _final(dinv, accp, h2, b2):
    return pl.pallas_call(
        _final_body,
        grid=(GRID,),
        in_specs=[pl.BlockSpec((BLK, FEAT), lambda i: (i, 0)),
                  pl.BlockSpec((NCORES, BLK, FEAT), lambda i: (0, i, 0)),
                  pl.BlockSpec((BLK, FEAT), lambda i: (i, 0)),
                  pl.BlockSpec((1, FEAT), lambda i: (0, 0))],
        out_specs=pl.BlockSpec((BLK, FEAT), lambda i: (i, 0)),
        out_shape=jax.ShapeDtypeStruct((NP, FEAT), _f32),
    )(dinv, accp, h2, b2)


def kernel(x, edge_index, W1, b1, W2, b2):
    xp = jnp.zeros((NP, FEAT), _f32).at[:N_NODES].set(x)
    pad = jnp.full((EP - E_EDGES,), N_NODES, jnp.int32)
    src = jnp.concatenate([edge_index[0], pad])
    dst = jnp.concatenate([edge_index[1], pad])
    ones128 = jnp.ones((K, FEAT), _f32)
    zeros128 = jnp.zeros((ZROWS, FEAT), _f32)

    degp = _deg_kernel(dst, ones128, zeros128)
    h1 = _mm(xp, W1)
    hs1, dinv = _scale(degp, h1)
    acc1 = _edge_kernel(hs1, src, dst, zeros128)
    h2, hs2 = _dense2(dinv, acc1, h1, b1.reshape(1, FEAT), W2)
    acc2 = _edge_kernel(hs2, src, dst, zeros128)
    y = _final(dinv, acc2, h2, b2.reshape(1, FEAT))
    return y[:N_NODES]
